# baseline (speedup 1.0000x reference)
import functools

import jax
import jax.numpy as jnp
from jax import lax
from jax.experimental import pallas as pl
from jax.experimental.pallas import tpu as pltpu
from jax.experimental.pallas import tpu_sc as plsc

_D = 768
_T_OUT = 64
_H_OUT = 32
_W_OUT = 32
_NC = 2
_NS = 16
_NW = _NC * _NS
_TPW = _T_OUT // _NW
_C = _D // 16


def _sc_body(scal_hbm, t_hbm, h_hbm, w_hbm, out_hbm,
             scal_v, idx_h, idx_w, h_rows, w_rows, trow, th,
             strip0, strip1, sem_in, sem0, sem1):
    cid = lax.axis_index("c")
    sid = lax.axis_index("s")
    wid = sid * _NC + cid

    pltpu.sync_copy(scal_hbm, scal_v)
    sv = scal_v[pl.ds(0, 16)]
    T = sv[0]
    nh = sv[1]
    nw = sv[2]

    for chunk in range(_H_OUT // 16):
        vec = lax.broadcasted_iota(jnp.int32, (16,), 0) + chunk * 16
        idx_h[pl.ds(chunk * 16, 16)] = jnp.minimum(vec, nh - 1)
        idx_w[pl.ds(chunk * 16, 16)] = jnp.minimum(vec, nw - 1)

    pltpu.async_copy(h_hbm.at[idx_h], h_rows, sem_in).wait()
    pltpu.async_copy(w_hbm.at[idx_w], w_rows, sem_in).wait()

    def strip_into(h, buf):
        def c_body(c, carry):
            thc = th[h, pl.ds(c * 16, 16)]

            def w_body(w, inner):
                buf[w, pl.ds(c * 16, 16)] = w_rows[w, pl.ds(c * 16, 16)] + thc
                return inner

            return lax.fori_loop(0, _W_OUT, w_body, carry)

        lax.fori_loop(0, _C, c_body, 0)

    for ti in range(_TPW):
        t = wid * _TPW + ti
        t_idx = jnp.minimum(t, T - 1)
        pltpu.sync_copy(t_hbm.at[pl.ds(t_idx, 1)], trow)

        def th_c_body(c, carry):
            tr = trow[0, pl.ds(c * 16, 16)]

            def th_h_body(h, inner):
                th[h, pl.ds(c * 16, 16)] = h_rows[h, pl.ds(c * 16, 16)] + tr
                return inner

            return lax.fori_loop(0, _H_OUT, th_h_body, carry)

        lax.fori_loop(0, _C, th_c_body, 0)

        base_t = t * (_H_OUT * _W_OUT)

        def pair_body(p, carry):
            h0 = p * 2

            @pl.when(p > 0)
            def _wait0():
                pltpu.make_async_copy(
                    strip0, out_hbm.at[pl.ds(base_t, _W_OUT)], sem0).wait()

            strip_into(h0, strip0)
            pltpu.async_copy(
                strip0, out_hbm.at[pl.ds(base_t + h0 * _W_OUT, _W_OUT)],
                sem0).start()

            @pl.when(p > 0)
            def _wait1():
                pltpu.make_async_copy(
                    strip1, out_hbm.at[pl.ds(base_t, _W_OUT)], sem1).wait()

            strip_into(h0 + 1, strip1)
            pltpu.async_copy(
                strip1, out_hbm.at[pl.ds(base_t + (h0 + 1) * _W_OUT, _W_OUT)],
                sem1).start()
            return carry

        lax.fori_loop(0, _H_OUT // 2, pair_body, 0)

        pltpu.make_async_copy(
            strip0, out_hbm.at[pl.ds(base_t, _W_OUT)], sem0).wait()
        pltpu.make_async_copy(
            strip1, out_hbm.at[pl.ds(base_t, _W_OUT)], sem1).wait()


def kernel(T, n_h, n_w, t_w, h_w, w_w):
    scal = jnp.zeros((16,), jnp.int32)
    scal = scal.at[0].set(jnp.asarray(T, jnp.int32))
    scal = scal.at[1].set(jnp.asarray(n_h, jnp.int32))
    scal = scal.at[2].set(jnp.asarray(n_w, jnp.int32))

    mesh = plsc.VectorSubcoreMesh(core_axis_name="c", subcore_axis_name="s")
    run = pl.kernel(
        _sc_body,
        out_type=jax.ShapeDtypeStruct((_T_OUT * _H_OUT * _W_OUT, _D),
                                      jnp.float32),
        mesh=mesh,
        scratch_types=[
            pltpu.VMEM((16,), jnp.int32),
            pltpu.VMEM((_H_OUT,), jnp.int32),
            pltpu.VMEM((_W_OUT,), jnp.int32),
            pltpu.VMEM((_H_OUT, _D), jnp.float32),
            pltpu.VMEM((_W_OUT, _D), jnp.float32),
            pltpu.VMEM((1, _D), jnp.float32),
            pltpu.VMEM((_H_OUT, _D), jnp.float32),
            pltpu.VMEM((_W_OUT, _D), jnp.float32),
            pltpu.VMEM((_W_OUT, _D), jnp.float32),
            pltpu.SemaphoreType.DMA,
            pltpu.SemaphoreType.DMA,
            pltpu.SemaphoreType.DMA,
        ],
    )
    out = run(scal, t_w, h_w, w_w)
    return out.reshape(_T_OUT, _H_OUT, _W_OUT, _D)


# speedup vs baseline: 5.2672x; 5.2672x over previous
import jax
import jax.numpy as jnp
from jax import lax
from jax.experimental import pallas as pl
from jax.experimental.pallas import tpu as pltpu
from jax.experimental.pallas import tpu_sc as plsc

_D = 768
_T_OUT = 64
_H_OUT = 32
_W_OUT = 32
_T_BLK = 2
_NC = 2


def _sc_gather_body(scal_hbm, t_hbm, h_hbm, w_hbm,
                    t_enc_hbm, h_enc_hbm, w_enc_hbm,
                    scal_v, idx_v, rows_v, sem):
    cid = lax.axis_index("c")
    sid = lax.axis_index("s")
    wid = sid * _NC + cid

    pltpu.sync_copy(scal_hbm, scal_v)
    sv = scal_v[pl.ds(0, 16)]

    def gather_table(n_rows, clamp, src_hbm, dst_hbm):
        for chunk in range(n_rows // 16):
            vec = lax.broadcasted_iota(jnp.int32, (16,), 0) + chunk * 16
            idx_v[pl.ds(chunk * 16, 16)] = jnp.minimum(vec, clamp - 1)
        pltpu.async_copy(
            src_hbm.at[idx_v.at[pl.ds(0, n_rows)]],
            rows_v.at[pl.ds(0, n_rows)], sem).wait()
        pltpu.sync_copy(rows_v.at[pl.ds(0, n_rows)], dst_hbm)

    @pl.when(wid == 0)
    def _():
        gather_table(_T_OUT, sv[0], t_hbm, t_enc_hbm)

    @pl.when(wid == 1)
    def _():
        gather_table(_H_OUT, sv[1], h_hbm, h_enc_hbm)

    @pl.when(wid == 2)
    def _():
        gather_table(_W_OUT, sv[2], w_hbm, w_enc_hbm)


def _tc_body(t_ref, h_ref, w_ref, out_ref):
    t = pl.program_id(0)
    h_rows = h_ref[...]
    w_rows = w_ref[...]
    for ti in range(_T_BLK):
        th = h_rows + t_ref[pl.ds(t * _T_BLK + ti, 1), :]
        for h in range(_H_OUT):
            out_ref[ti, h] = th[h:h + 1, :] + w_rows


def kernel(T, n_h, n_w, t_w, h_w, w_w):
    scal = jnp.zeros((16,), jnp.int32)
    scal = scal.at[0].set(jnp.asarray(T, jnp.int32))
    scal = scal.at[1].set(jnp.asarray(n_h, jnp.int32))
    scal = scal.at[2].set(jnp.asarray(n_w, jnp.int32))

    mesh = plsc.VectorSubcoreMesh(core_axis_name="c", subcore_axis_name="s")
    t_enc, h_enc, w_enc = pl.kernel(
        _sc_gather_body,
        out_type=(
            jax.ShapeDtypeStruct((_T_OUT, _D), jnp.float32),
            jax.ShapeDtypeStruct((_H_OUT, _D), jnp.float32),
            jax.ShapeDtypeStruct((_W_OUT, _D), jnp.float32),
        ),
        mesh=mesh,
        scratch_types=[
            pltpu.VMEM((16,), jnp.int32),
            pltpu.VMEM((_T_OUT,), jnp.int32),
            pltpu.VMEM((_T_OUT, _D), jnp.float32),
            pltpu.SemaphoreType.DMA,
        ],
    )(scal, t_w, h_w, w_w)

    return pl.pallas_call(
        _tc_body,
        grid=(_T_OUT // _T_BLK,),
        in_specs=[
            pl.BlockSpec((_T_OUT, _D), lambda i: (0, 0)),
            pl.BlockSpec((_H_OUT, _D), lambda i: (0, 0)),
            pl.BlockSpec((_W_OUT, _D), lambda i: (0, 0)),
        ],
        out_specs=pl.BlockSpec((_T_BLK, _H_OUT, _W_OUT, _D),
                               lambda i: (i, 0, 0, 0)),
        out_shape=jax.ShapeDtypeStruct((_T_OUT, _H_OUT, _W_OUT, _D),
                                       jnp.float32),
        compiler_params=pltpu.CompilerParams(
            dimension_semantics=("arbitrary",)),
    )(t_enc, h_enc, w_enc)
